# Initial kernel scaffold; baseline (speedup 1.0000x reference)
import jax
import jax.numpy as jnp
from jax.experimental import pallas as pl


def kernel(authors, papers, edge_index, author_emb, paper_emb):
    raise NotImplementedError("write your pallas kernel here")


# speedup vs baseline: 3.9479x; 3.9479x over previous
import functools

import jax
import jax.numpy as jnp
from jax import lax
from jax.experimental import pallas as pl
from jax.experimental.pallas import tpu as pltpu
from jax.experimental.pallas import tpu_sc as plsc

NUM_NODES = 50000
EMB_DIM = 64
NUM_EDGES = 800000
BATCH = 16384

NC = 2
NS = 16
L = 16

HALF = 25088
N_PAD = 2 * HALF
GARB = 512
SP_ROWS = HALF + GARB
ZROWS = SP_ROWS // NS
NRM = HALF // NS
NRM_C = 112
E = 128
NCHUNKS = NUM_EDGES // E

_mesh = lambda: plsc.VectorSubcoreMesh(
    core_axis_name="c", subcore_axis_name="s", num_cores=NC, num_subcores=NS)
_SC_PARAMS = pltpu.CompilerParams(
    needs_layout_passes=False, use_tc_tiling_on_sc=False)


def _dst_local(dst16, base, off):
    m = (dst16 >= base) & (dst16 < base + HALF)
    garb = HALF + ((off + lax.iota(jnp.int32, L)) & (GARB - 1))
    return jnp.where(m, dst16 - base, garb)


def _deg_body(dst_hbm, inv_hbm, deg_sp, zer_v, dst_v, idx_v, one_v, wrk_v):
    core = lax.axis_index("c")
    sub = lax.axis_index("s")
    base = core * HALF

    def _zero(i, c):
        zer_v[pl.ds(i * L, L)] = jnp.zeros((L,), jnp.float32)
        return c
    lax.fori_loop(0, ZROWS // L, _zero, 0)
    pltpu.sync_copy(zer_v, deg_sp.at[pl.ds(sub * ZROWS, ZROWS)])

    def _ones(i, c):
        one_v[pl.ds(i * L, L)] = jnp.full((L,), 1.0, jnp.float32)
        return c
    lax.fori_loop(0, E // L, _ones, 0)
    plsc.subcore_barrier()

    nt = (NCHUNKS - sub + NS - 1) // NS

    def _chunk(j, c):
        off = (sub + j * NS) * E
        pltpu.sync_copy(dst_hbm.at[pl.ds(off, E)], dst_v)
        for k in range(E // L):
            d = dst_v[pl.ds(k * L, L)]
            idx_v[pl.ds(k * L, L)] = _dst_local(d, base, off + k * L)
        pltpu.sync_copy(one_v, deg_sp.at[idx_v], add=True)
        return c
    lax.fori_loop(0, nt, _chunk, 0)
    plsc.subcore_barrier()

    r0 = sub * NRM
    for q in range(NRM // NRM_C):
        pltpu.sync_copy(deg_sp.at[pl.ds(r0 + q * NRM_C, NRM_C)], wrk_v)

        def _inv(i, c):
            v = wrk_v[pl.ds(i * L, L)]
            wrk_v[pl.ds(i * L, L)] = 1.0 / jnp.maximum(v, 1.0)
            return c
        lax.fori_loop(0, NRM_C // L, _inv, 0)
        pltpu.sync_copy(wrk_v, inv_hbm.at[pl.ds(base + r0 + q * NRM_C, NRM_C)])


def _conv_body(emb_hbm, src_hbm, dst_hbm, inv_hbm, out_hbm,
               agg_sp, zer_v, src_v, dst_v, idx_v, row_v, nrm_v, inv_v, sem):
    core = lax.axis_index("c")
    sub = lax.axis_index("s")
    base = core * HALF

    zc = 100

    def _zero(r, c):
        for k in range(EMB_DIM // L):
            zer_v[r, pl.ds(k * L, L)] = jnp.zeros((L,), jnp.float32)
        return c
    lax.fori_loop(0, zc, _zero, 0)
    for q in range(ZROWS // zc):
        pltpu.sync_copy(zer_v, agg_sp.at[pl.ds(sub * ZROWS + q * zc, zc)])
    plsc.subcore_barrier()

    nt = (NCHUNKS - sub + NS - 1) // NS

    def _chunk(j, c):
        off = (sub + j * NS) * E
        pltpu.sync_copy(src_hbm.at[pl.ds(off, E)], src_v)
        pltpu.sync_copy(dst_hbm.at[pl.ds(off, E)], dst_v)
        for k in range(E // L):
            d = dst_v[pl.ds(k * L, L)]
            idx_v[pl.ds(k * L, L)] = _dst_local(d, base, off + k * L)
        pltpu.async_copy(emb_hbm.at[src_v], row_v, sem).wait()
        pltpu.sync_copy(row_v, agg_sp.at[idx_v], add=True)
        return c
    lax.fori_loop(0, nt, _chunk, 0)
    plsc.subcore_barrier()

    r0 = sub * NRM
    for q in range(NRM // NRM_C):
        pltpu.sync_copy(agg_sp.at[pl.ds(r0 + q * NRM_C, NRM_C)], nrm_v)
        pltpu.sync_copy(inv_hbm.at[pl.ds(base + r0 + q * NRM_C, NRM_C)], inv_v)

        def _scale(r, c):
            s = plsc.load_gather(inv_v, [jnp.full((L,), r, jnp.int32)])
            for k in range(EMB_DIM // L):
                nrm_v[r, pl.ds(k * L, L)] = nrm_v[r, pl.ds(k * L, L)] * s
            return c
        lax.fori_loop(0, NRM_C, _scale, 0)
        pltpu.sync_copy(nrm_v, out_hbm.at[pl.ds(base + r0 + q * NRM_C, NRM_C)])


def _gather_body(a_hbm, g1_hbm, g2_hbm, p_hbm, au_hbm, pa_hbm,
                 oa_hbm, op_hbm, idx_v, acc_v, tmp_v, sem):
    core = lax.axis_index("c")
    sub = lax.axis_index("s")
    wid = sub * NC + core
    per_w = BATCH // (NC * NS)

    def _acc_add(r, c):
        for k in range(EMB_DIM // L):
            acc_v[r, pl.ds(k * L, L)] = (
                acc_v[r, pl.ds(k * L, L)] + tmp_v[r, pl.ds(k * L, L)])
        return c

    for q in range(per_w // E):
        b0 = wid * per_w + q * E
        pltpu.sync_copy(au_hbm.at[pl.ds(b0, E)], idx_v)
        pltpu.async_copy(a_hbm.at[idx_v], acc_v, sem).wait()
        pltpu.async_copy(g1_hbm.at[idx_v], tmp_v, sem).wait()
        lax.fori_loop(0, E, _acc_add, 0)
        pltpu.async_copy(g2_hbm.at[idx_v], tmp_v, sem).wait()
        lax.fori_loop(0, E, _acc_add, 0)
        pltpu.sync_copy(acc_v, oa_hbm.at[pl.ds(b0, E)])

        pltpu.sync_copy(pa_hbm.at[pl.ds(b0, E)], idx_v)
        pltpu.async_copy(p_hbm.at[idx_v], tmp_v, sem).wait()
        pltpu.sync_copy(tmp_v, op_hbm.at[pl.ds(b0, E)])


def _predict_body(a_ref, p_ref, o_ref):
    o_ref[...] = jax.nn.sigmoid(jnp.sum(a_ref[...] * p_ref[...], axis=1))


@jax.jit
def _run(authors, papers, src, dst, author_emb, paper_emb):
    f32 = jnp.float32
    deg_kernel = pl.kernel(
        _deg_body,
        out_type=jax.ShapeDtypeStruct((N_PAD,), f32),
        mesh=_mesh(),
        compiler_params=_SC_PARAMS,
        scratch_types=[
            pltpu.VMEM_SHARED((SP_ROWS,), f32),
            pltpu.VMEM((ZROWS,), f32),
            pltpu.VMEM((E,), jnp.int32),
            pltpu.VMEM((E,), jnp.int32),
            pltpu.VMEM((E,), f32),
            pltpu.VMEM((NRM_C,), f32),
        ],
    )
    inv = deg_kernel(dst)

    def conv(emb):
        return pl.kernel(
            _conv_body,
            out_type=jax.ShapeDtypeStruct((N_PAD, EMB_DIM), f32),
            mesh=_mesh(),
            compiler_params=_SC_PARAMS,
            scratch_types=[
                pltpu.VMEM_SHARED((SP_ROWS, EMB_DIM), f32),
                pltpu.VMEM((100, EMB_DIM), f32),
                pltpu.VMEM((E,), jnp.int32),
                pltpu.VMEM((E,), jnp.int32),
                pltpu.VMEM((E,), jnp.int32),
                pltpu.VMEM((E, EMB_DIM), f32),
                pltpu.VMEM((NRM_C, EMB_DIM), f32),
                pltpu.VMEM((NRM_C,), f32),
                pltpu.SemaphoreType.DMA,
            ],
        )(emb, src, dst, inv)

    g1 = conv(author_emb)
    g2 = conv(g1)

    gather_kernel = pl.kernel(
        _gather_body,
        out_type=[
            jax.ShapeDtypeStruct((BATCH, EMB_DIM), f32),
            jax.ShapeDtypeStruct((BATCH, EMB_DIM), f32),
        ],
        mesh=_mesh(),
        compiler_params=_SC_PARAMS,
        scratch_types=[
            pltpu.VMEM((E,), jnp.int32),
            pltpu.VMEM((E, EMB_DIM), f32),
            pltpu.VMEM((E, EMB_DIM), f32),
            pltpu.SemaphoreType.DMA,
        ],
    )
    la, lp = gather_kernel(author_emb, g1, g2, paper_emb, authors, papers)

    blk = 2048
    pred = pl.pallas_call(
        _predict_body,
        grid=(BATCH // blk,),
        in_specs=[
            pl.BlockSpec((blk, EMB_DIM), lambda i: (i, 0)),
            pl.BlockSpec((blk, EMB_DIM), lambda i: (i, 0)),
        ],
        out_specs=pl.BlockSpec((blk,), lambda i: (i,)),
        out_shape=jax.ShapeDtypeStruct((BATCH,), f32),
    )(la, lp)
    return pred, la, lp


def kernel(authors, papers, edge_index, author_emb, paper_emb):
    authors = authors.astype(jnp.int32)
    papers = papers.astype(jnp.int32)
    src = edge_index[0].astype(jnp.int32)
    dst = edge_index[1].astype(jnp.int32)
    return _run(authors, papers, src, dst, author_emb, paper_emb)


# speedup vs baseline: 9.7287x; 2.4643x over previous
import jax
import jax.numpy as jnp
from jax import lax
from jax.experimental import pallas as pl
from jax.experimental.pallas import tpu as pltpu
from jax.experimental.pallas import tpu_sc as plsc

NUM_NODES = 50000
EMB_DIM = 64
NUM_EDGES = 800000
BATCH = 16384

NC = 2
NS = 16
L = 16

HALF = 25088
N_PAD = 2 * HALF
GARB = 512
SP_ROWS = HALF + GARB
ZROWS = SP_ROWS // NS
NRM = HALF // NS
NRM_C = 112
E = 128
NT = 391
NCHUNKS_P = NT * NS
PAD_E = NCHUNKS_P * E - NUM_EDGES
DST_PAD = N_PAD

_mesh = lambda: plsc.VectorSubcoreMesh(
    core_axis_name="c", subcore_axis_name="s", num_cores=NC, num_subcores=NS)
_SC_PARAMS = pltpu.CompilerParams(
    needs_layout_passes=False, use_tc_tiling_on_sc=False)


def _dst_local(dst16, base, off):
    m = (dst16 >= base) & (dst16 < base + HALF)
    garb = HALF + ((off + lax.iota(jnp.int32, L)) & (GARB - 1))
    return jnp.where(m, dst16 - base, garb)


def _make_conv_body(with_deg):
    def body(emb_hbm, src_hbm, dst_hbm, *rest):
        if with_deg:
            (out_hbm, inv_hbm, agg_sp, deg_sp,
             src0, src1, dst0, dst1, idx0, idx1, row0, row1,
             one_v, zer1, nrm_v, wrk_v,
             se0, se1, sg0, sg1, ss0, ss1, sd0, sd1) = rest
        else:
            (inv_hbm, out_hbm, agg_sp,
             src0, src1, dst0, dst1, idx0, idx1, row0, row1,
             zer1, nrm_v, wrk_v,
             se0, se1, sg0, sg1, ss0, ss1) = rest
            deg_sp = one_v = sd0 = sd1 = None
        srcs, dsts, idxs, rows = (src0, src1), (dst0, dst1), (idx0, idx1), (row0, row1)
        sem_e, sem_g, sem_s, sem_d = (se0, se1), (sg0, sg1), (ss0, ss1), (sd0, sd1)

        core = lax.axis_index("c")
        sub = lax.axis_index("s")
        base = core * HALF

        def _zrow(r, c):
            for k in range(EMB_DIM // L):
                row0[r, pl.ds(k * L, L)] = jnp.zeros((L,), jnp.float32)
            return c
        lax.fori_loop(0, E, _zrow, 0)
        z0 = sub * ZROWS
        for q in range(ZROWS // E):
            pltpu.sync_copy(row0, agg_sp.at[pl.ds(z0 + q * E, E)])
        pltpu.sync_copy(row0.at[pl.ds(0, ZROWS - (ZROWS // E) * E)],
                        agg_sp.at[pl.ds(z0 + (ZROWS // E) * E,
                                        ZROWS - (ZROWS // E) * E)])
        if with_deg:
            def _z1(i, c):
                zer1[pl.ds(i * L, L)] = jnp.zeros((L,), jnp.float32)
                return c
            lax.fori_loop(0, 160 // L, _z1, 0)
            for q in range(ZROWS // 160):
                pltpu.sync_copy(zer1, deg_sp.at[pl.ds(z0 + q * 160, 160)])

            def _ones(i, c):
                one_v[pl.ds(i * L, L)] = jnp.full((L,), 1.0, jnp.float32)
                return c
            lax.fori_loop(0, E // L, _ones, 0)
        plsc.subcore_barrier()

        def _e_start(j, s):
            off = (sub + j * NS) * E
            pltpu.async_copy(src_hbm.at[pl.ds(off, E)], srcs[s], sem_e[s])
            pltpu.async_copy(dst_hbm.at[pl.ds(off, E)], dsts[s], sem_e[s])

        def _e_wait(s):
            pltpu.make_async_copy(src_hbm.at[pl.ds(0, E)], srcs[s], sem_e[s]).wait()
            pltpu.make_async_copy(dst_hbm.at[pl.ds(0, E)], dsts[s], sem_e[s]).wait()

        def _x(j, s):
            off = (sub + j * NS) * E
            for k in range(E // L):
                d = dsts[s][pl.ds(k * L, L)]
                idxs[s][pl.ds(k * L, L)] = _dst_local(d, base, off + k * L)

        def _g_start(s):
            pltpu.async_copy(emb_hbm.at[srcs[s]], rows[s], sem_g[s])

        def _g_wait(s):
            pltpu.make_async_copy(emb_hbm.at[srcs[s]], rows[s], sem_g[s]).wait()

        def _s_start(s):
            pltpu.async_copy(rows[s], agg_sp.at[idxs[s]], sem_s[s], add=True)
            if with_deg:
                pltpu.async_copy(one_v, deg_sp.at[idxs[s]], sem_d[s], add=True)

        def _s_wait(s):
            pltpu.make_async_copy(rows[s], agg_sp.at[idxs[s]], sem_s[s]).wait()
            if with_deg:
                pltpu.make_async_copy(one_v, deg_sp.at[idxs[s]], sem_d[s]).wait()

        _e_start(0, 0)
        _e_wait(0)
        _x(0, 0)
        _g_start(0)
        _e_start(1, 1)
        _e_wait(1)
        _x(1, 1)
        _g_start(1)
        _g_wait(0)
        _e_start(2, 0)
        _s_start(0)

        def _full(j, s):
            o = 1 - s
            _e_wait(s)
            _s_wait(s)
            _x(j, s)
            _g_start(s)
            _g_wait(o)
            _e_start(j + 1, o)
            _s_start(o)

        def _pair(p, c):
            _full(2 + 2 * p, 0)
            _full(3 + 2 * p, 1)
            return c
        lax.fori_loop(0, (NT - 3) // 2, _pair, 0)

        _e_wait(0)
        _s_wait(0)
        _x(NT - 1, 0)
        _g_start(0)
        _g_wait(1)
        _s_start(1)
        _g_wait(0)
        _s_wait(1)
        _s_start(0)
        _s_wait(0)
        plsc.subcore_barrier()

        r0 = sub * NRM
        for q in range(NRM // NRM_C):
            rq = r0 + q * NRM_C
            if with_deg:
                pltpu.sync_copy(deg_sp.at[pl.ds(rq, NRM_C)], wrk_v)

                def _inv(i, c):
                    v = wrk_v[pl.ds(i * L, L)]
                    wrk_v[pl.ds(i * L, L)] = 1.0 / jnp.maximum(v, 1.0)
                    return c
                lax.fori_loop(0, NRM_C // L, _inv, 0)
                pltpu.sync_copy(wrk_v, inv_hbm.at[pl.ds(base + rq, NRM_C)])
            else:
                pltpu.sync_copy(inv_hbm.at[pl.ds(base + rq, NRM_C)], wrk_v)
            pltpu.sync_copy(agg_sp.at[pl.ds(rq, NRM_C)], nrm_v)

            def _scale(r, c):
                s = plsc.load_gather(wrk_v, [jnp.full((L,), r, jnp.int32)])
                for k in range(EMB_DIM // L):
                    nrm_v[r, pl.ds(k * L, L)] = nrm_v[r, pl.ds(k * L, L)] * s
                return c
            lax.fori_loop(0, NRM_C, _scale, 0)
            pltpu.sync_copy(nrm_v, out_hbm.at[pl.ds(base + rq, NRM_C)])
    return body


def _gather_body(a_hbm, g1_hbm, g2_hbm, p_hbm, au_hbm, pa_hbm,
                 oa_hbm, op_hbm, idx_v, acc_v, tmp_v, sem):
    core = lax.axis_index("c")
    sub = lax.axis_index("s")
    wid = sub * NC + core
    per_w = BATCH // (NC * NS)

    def _acc_add(r, c):
        for k in range(EMB_DIM // L):
            acc_v[r, pl.ds(k * L, L)] = (
                acc_v[r, pl.ds(k * L, L)] + tmp_v[r, pl.ds(k * L, L)])
        return c

    for q in range(per_w // E):
        b0 = wid * per_w + q * E
        pltpu.sync_copy(au_hbm.at[pl.ds(b0, E)], idx_v)
        pltpu.async_copy(a_hbm.at[idx_v], acc_v, sem).wait()
        pltpu.async_copy(g1_hbm.at[idx_v], tmp_v, sem).wait()
        lax.fori_loop(0, E, _acc_add, 0)
        pltpu.async_copy(g2_hbm.at[idx_v], tmp_v, sem).wait()
        lax.fori_loop(0, E, _acc_add, 0)
        pltpu.sync_copy(acc_v, oa_hbm.at[pl.ds(b0, E)])

        pltpu.sync_copy(pa_hbm.at[pl.ds(b0, E)], idx_v)
        pltpu.async_copy(p_hbm.at[idx_v], tmp_v, sem).wait()
        pltpu.sync_copy(tmp_v, op_hbm.at[pl.ds(b0, E)])


def _predict_body(a_ref, p_ref, o_ref):
    o_ref[...] = jax.nn.sigmoid(jnp.sum(a_ref[...] * p_ref[...], axis=1))


@jax.jit
def _run(authors, papers, src, dst, author_emb, paper_emb):
    f32 = jnp.float32
    i32 = jnp.int32

    src = jnp.concatenate([src, jnp.zeros((PAD_E,), i32)])
    dst = jnp.concatenate([dst, jnp.full((PAD_E,), DST_PAD, i32)])

    def pipe_scratch():
        return [
            pltpu.VMEM((E,), i32), pltpu.VMEM((E,), i32),
            pltpu.VMEM((E,), i32), pltpu.VMEM((E,), i32),
            pltpu.VMEM((E,), i32), pltpu.VMEM((E,), i32),
            pltpu.VMEM((E, EMB_DIM), f32), pltpu.VMEM((E, EMB_DIM), f32),
        ]

    conv1 = pl.kernel(
        _make_conv_body(True),
        out_type=[
            jax.ShapeDtypeStruct((N_PAD, EMB_DIM), f32),
            jax.ShapeDtypeStruct((N_PAD,), f32),
        ],
        mesh=_mesh(),
        compiler_params=_SC_PARAMS,
        scratch_types=(
            [pltpu.VMEM_SHARED((SP_ROWS, EMB_DIM), f32),
             pltpu.VMEM_SHARED((SP_ROWS,), f32)]
            + pipe_scratch()
            + [pltpu.VMEM((E,), f32),
               pltpu.VMEM((160,), f32),
               pltpu.VMEM((NRM_C, EMB_DIM), f32),
               pltpu.VMEM((NRM_C,), f32)]
            + [pltpu.SemaphoreType.DMA] * 8
        ),
    )
    g1, inv = conv1(author_emb, src, dst)

    conv2 = pl.kernel(
        _make_conv_body(False),
        out_type=jax.ShapeDtypeStruct((N_PAD, EMB_DIM), f32),
        mesh=_mesh(),
        compiler_params=_SC_PARAMS,
        scratch_types=(
            [pltpu.VMEM_SHARED((SP_ROWS, EMB_DIM), f32)]
            + pipe_scratch()
            + [pltpu.VMEM((160,), f32),
               pltpu.VMEM((NRM_C, EMB_DIM), f32),
               pltpu.VMEM((NRM_C,), f32)]
            + [pltpu.SemaphoreType.DMA] * 6
        ),
    )
    g2 = conv2(g1, src, dst, inv)

    gather_kernel = pl.kernel(
        _gather_body,
        out_type=[
            jax.ShapeDtypeStruct((BATCH, EMB_DIM), f32),
            jax.ShapeDtypeStruct((BATCH, EMB_DIM), f32),
        ],
        mesh=_mesh(),
        compiler_params=_SC_PARAMS,
        scratch_types=[
            pltpu.VMEM((E,), i32),
            pltpu.VMEM((E, EMB_DIM), f32),
            pltpu.VMEM((E, EMB_DIM), f32),
            pltpu.SemaphoreType.DMA,
        ],
    )
    la, lp = gather_kernel(author_emb, g1, g2, paper_emb, authors, papers)

    blk = 2048
    pred = pl.pallas_call(
        _predict_body,
        grid=(BATCH // blk,),
        in_specs=[
            pl.BlockSpec((blk, EMB_DIM), lambda i: (i, 0)),
            pl.BlockSpec((blk, EMB_DIM), lambda i: (i, 0)),
        ],
        out_specs=pl.BlockSpec((blk,), lambda i: (i,)),
        out_shape=jax.ShapeDtypeStruct((BATCH,), f32),
    )(la, lp)
    return pred, la, lp


def kernel(authors, papers, edge_index, author_emb, paper_emb):
    authors = authors.astype(jnp.int32)
    papers = papers.astype(jnp.int32)
    src = edge_index[0].astype(jnp.int32)
    dst = edge_index[1].astype(jnp.int32)
    return _run(authors, papers, src, dst, author_emb, paper_emb)
